# baseline (speedup 1.0000x reference)
import jax
import jax.numpy as jnp
from jax.experimental import pallas as pl
from jax.experimental.pallas import tpu as pltpu


def _se_fused_body(x_ref, w1_ref, b1_ref, w2_ref, b2_ref, o_ref):
    x = x_ref[...]
    pooled = jnp.mean(x, axis=2)
    h = jnp.dot(pooled, w1_ref[...], preferred_element_type=jnp.float32)
    h = jnp.maximum(h + b1_ref[...], 0.0)
    g = jnp.dot(h, w2_ref[...], preferred_element_type=jnp.float32)
    g = jax.nn.sigmoid(g + b2_ref[...])
    o_ref[...] = (x * g[:, :, None].astype(x.dtype)).astype(o_ref.dtype)


def kernel(x, w1, b1, w2, b2):
    B, C, H, W = x.shape
    HW = H * W
    hidden = w1.shape[1]
    itemsize = jnp.dtype(x.dtype).itemsize

    lanes = -(-HW // 128) * 128
    bytes_per_sample_padded = C * lanes * itemsize
    max_bt = (40 * 1024 * 1024) // (4 * bytes_per_sample_padded)
    bt = int(max(1, min(max_bt, pl.cdiv(B, 2), 32)))
    grid = (int(pl.cdiv(B, bt)),)

    x3 = x.reshape(B, C, HW)
    block = (bt, C, HW)

    out = pl.pallas_call(
        _se_fused_body,
        out_shape=jax.ShapeDtypeStruct((B, C, HW), x.dtype),
        grid=grid,
        in_specs=[
            pl.BlockSpec(block, lambda b: (b, 0, 0)),
            pl.BlockSpec((C, hidden), lambda b: (0, 0)),
            pl.BlockSpec((1, hidden), lambda b: (0, 0)),
            pl.BlockSpec((hidden, C), lambda b: (0, 0)),
            pl.BlockSpec((1, C), lambda b: (0, 0)),
        ],
        out_specs=pl.BlockSpec(block, lambda b: (b, 0, 0)),
        compiler_params=pltpu.CompilerParams(
            dimension_semantics=("parallel",),
            vmem_limit_bytes=56 * 1024 * 1024,
        ),
        cost_estimate=pl.CostEstimate(
            flops=3 * B * C * HW + 4 * B * C * hidden,
            transcendentals=B * C,
            bytes_accessed=2 * B * C * HW * itemsize,
        ),
    )(x3, w1, b1.reshape(1, hidden), w2, b2.reshape(1, C))

    return out.reshape(B, C, H, W)
